# baseline (speedup 1.0000x reference)
import jax
import jax.numpy as jnp
from jax import lax
from jax.experimental import pallas as pl
from jax.experimental.pallas import tpu as pltpu
from jax.experimental.pallas import tpu_sc as plsc

_B4, _S, _D = 4096, 26, 64
_NC, _NS = 2, 16
_NW = _NC * _NS
_RPW = _B4 // _NW
_NBUF = 4
_NR = _RPW // _NBUF


def _gather_body(x_hbm, table_hbm, out_hbm, idx_v, rows_v,
                 gs0, gs1, gs2, gs3, ss0, ss1, ss2, ss3):
    gsems = (gs0, gs1, gs2, gs3)
    ssems = (ss0, ss1, ss2, ss3)
    wid = lax.axis_index("s") * _NC + lax.axis_index("c")
    row0 = wid * _RPW
    pltpu.sync_copy(x_hbm.at[pl.ds(row0, _RPW)], idx_v)

    def g_copy(j, b):
        return pltpu.make_async_copy(
            table_hbm.at[idx_v.at[j]], rows_v.at[b], gsems[b])

    def s_copy(j, b):
        return pltpu.make_async_copy(
            rows_v.at[b], out_hbm.at[row0 + j], ssems[b])

    for b in range(_NBUF):
        g_copy(b, b).start()

    def round_body(r, carry):
        for b in range(_NBUF):
            g_copy((r - 1) * _NBUF + b, b).wait()
            s_copy((r - 1) * _NBUF + b, b).start()
        for b in range(_NBUF):
            s_copy((r - 1) * _NBUF + b, b).wait()
            g_copy(r * _NBUF + b, b).start()
        return carry

    lax.fori_loop(1, _NR, round_body, 0, unroll=False)

    for b in range(_NBUF):
        g_copy((_NR - 1) * _NBUF + b, b).wait()
        s_copy((_NR - 1) * _NBUF + b, b).start()
    for b in range(_NBUF):
        s_copy((_NR - 1) * _NBUF + b, b).wait()


@jax.jit
def _gather(x, table):
    mesh = plsc.VectorSubcoreMesh(core_axis_name="c", subcore_axis_name="s")
    f = pl.kernel(
        _gather_body,
        out_type=jax.ShapeDtypeStruct((_B4, _S, _D), jnp.float32),
        mesh=mesh,
        scratch_types=[
            pltpu.VMEM((_RPW, _S), jnp.int32),
            pltpu.VMEM((_NBUF, _S, _D), jnp.float32),
        ] + [pltpu.SemaphoreType.DMA] * (2 * _NBUF),
        compiler_params=pltpu.CompilerParams(use_tc_tiling_on_sc=False),
    )
    return f(x, table)


def kernel(x, weight):
    return _gather(x.astype(jnp.int32), weight)


# speedup vs baseline: 1.0664x; 1.0664x over previous
import jax
import jax.numpy as jnp
from jax import lax
from jax.experimental import pallas as pl
from jax.experimental.pallas import tpu as pltpu
from jax.experimental.pallas import tpu_sc as plsc

_B4, _S, _D = 4096, 26, 64
_DP = 128
_NC, _NS = 2, 16
_NW = _NC * _NS
_RPW = _B4 // _NW
_NBUF = 4
_NR = _RPW // _NBUF


def _gather_body(x_hbm, table_hbm, out_hbm, idx_v, rows_v,
                 gs0, gs1, gs2, gs3, ss0, ss1, ss2, ss3):
    gsems = (gs0, gs1, gs2, gs3)
    ssems = (ss0, ss1, ss2, ss3)
    wid = lax.axis_index("s") * _NC + lax.axis_index("c")
    row0 = wid * _RPW
    pltpu.sync_copy(x_hbm.at[pl.ds(row0, _RPW)], idx_v)

    def g_copy(j, b):
        return pltpu.make_async_copy(
            table_hbm.at[idx_v.at[j]], rows_v.at[b], gsems[b])

    def s_copy(j, b):
        return pltpu.make_async_copy(
            rows_v.at[b].at[:, pl.ds(0, _D)], out_hbm.at[row0 + j],
            ssems[b])

    for b in range(_NBUF):
        g_copy(b, b).start()

    def round_body(r, carry):
        for b in range(_NBUF):
            g_copy((r - 1) * _NBUF + b, b).wait()
            s_copy((r - 1) * _NBUF + b, b).start()
        for b in range(_NBUF):
            s_copy((r - 1) * _NBUF + b, b).wait()
            g_copy(r * _NBUF + b, b).start()
        return carry

    lax.fori_loop(1, _NR, round_body, 0, unroll=False)

    for b in range(_NBUF):
        g_copy((_NR - 1) * _NBUF + b, b).wait()
        s_copy((_NR - 1) * _NBUF + b, b).start()
    for b in range(_NBUF):
        s_copy((_NR - 1) * _NBUF + b, b).wait()


@jax.jit
def _gather(x, table):
    mesh = plsc.VectorSubcoreMesh(core_axis_name="c", subcore_axis_name="s")
    f = pl.kernel(
        _gather_body,
        out_type=jax.ShapeDtypeStruct((_B4, _S, _D), jnp.float32),
        mesh=mesh,
        scratch_types=[
            pltpu.VMEM((_RPW, _S), jnp.int32),
            pltpu.VMEM((_NBUF, _S, _DP), jnp.float32),
        ] + [pltpu.SemaphoreType.DMA] * (2 * _NBUF),
        compiler_params=pltpu.CompilerParams(use_tc_tiling_on_sc=False),
    )
    return f(x, table)


def kernel(x, weight):
    wp = jnp.pad(weight, ((0, 0), (0, _DP - _D)))
    return _gather(x.astype(jnp.int32), wp)


# speedup vs baseline: 1.0948x; 1.0266x over previous
import jax
import jax.numpy as jnp
from jax import lax
from jax.experimental import pallas as pl
from jax.experimental.pallas import tpu as pltpu
from jax.experimental.pallas import tpu_sc as plsc

_B4, _S, _D = 4096, 26, 64
_NC, _NS = 2, 16
_NW = _NC * _NS
_RPW = _B4 // _NW
_NBUF = 4
_NR = _RPW // _NBUF


def _gather_body(x_hbm, table_hbm, out_hbm, idx_v, rows_v,
                 gs0, gs1, gs2, gs3, ss0, ss1, ss2, ss3):
    gsems = (gs0, gs1, gs2, gs3)
    ssems = (ss0, ss1, ss2, ss3)
    wid = lax.axis_index("s") * _NC + lax.axis_index("c")
    row0 = wid * _RPW
    pltpu.sync_copy(x_hbm.at[pl.ds(row0, _RPW)], idx_v)

    def g_copy(j, b):
        return pltpu.make_async_copy(
            table_hbm.at[idx_v.at[j]], rows_v.at[b], gsems[b])

    def s_copy(j, b):
        return pltpu.make_async_copy(
            rows_v.at[b], out_hbm.at[row0 + j], ssems[b])

    for b in range(_NBUF):
        g_copy(b, b).start()

    def round_body(r, carry):
        for b in range(_NBUF):
            g_copy((r - 1) * _NBUF + b, b).wait()
            s_copy((r - 1) * _NBUF + b, b).start()
        for b in range(_NBUF):
            s_copy((r - 1) * _NBUF + b, b).wait()
            g_copy(r * _NBUF + b, b).start()
        return carry

    lax.fori_loop(1, _NR, round_body, 0, unroll=False)

    for b in range(_NBUF):
        g_copy((_NR - 1) * _NBUF + b, b).wait()
        s_copy((_NR - 1) * _NBUF + b, b).start()
    for b in range(_NBUF):
        s_copy((_NR - 1) * _NBUF + b, b).wait()


@jax.jit
def _gather(x2, table2):
    mesh = plsc.VectorSubcoreMesh(core_axis_name="c", subcore_axis_name="s")
    f = pl.kernel(
        _gather_body,
        out_type=jax.ShapeDtypeStruct((_B4, _S, _D), jnp.float32),
        mesh=mesh,
        scratch_types=[
            pltpu.VMEM((_RPW, _S), jnp.int32),
            pltpu.VMEM((_NBUF, _S, _D), jnp.float32),
        ] + [pltpu.SemaphoreType.DMA] * (2 * _NBUF),
        compiler_params=pltpu.CompilerParams(use_tc_tiling_on_sc=False),
    )
    return f(x2, table2)


def kernel(x, weight):
    wp = jnp.pad(weight, ((0, 0), (0, _D))).reshape(2 * weight.shape[0], _D)
    x2 = x.astype(jnp.int32) * 2
    return _gather(x2, wp)
